# baseline (speedup 1.0000x reference)
import functools

import jax
import jax.numpy as jnp
from jax import lax
from jax.experimental import pallas as pl
from jax.experimental.pallas import tpu as pltpu
from jax.experimental.pallas import tpu_sc as plsc

N = 8 * 224 * 224
ROWS = N // 128
BLOCK_ROWS = 56
GRID = ROWS // BLOCK_ROWS
NUM_BINS = 313
PAD_BINS = 320
GROUP = 16
BIG = 3.0e38


def _argmin_l2_body(centers_ref, ta_ref, tb_ref, pa_ref, pb_ref, bins_ref, l2_ref,
                    coef_ref):
    @pl.when(pl.program_id(0) == 0)
    def _prep():
        def pbody(j, carry):
            c_a = centers_ref[j, 0]
            c_b = centers_ref[j, 1]
            coef_ref[j, 0] = c_a * -256.0
            coef_ref[j, 1] = c_b * -256.0
            coef_ref[j, 2] = c_a * c_a + c_b * c_b
            return carry
        lax.fori_loop(0, NUM_BINS, pbody, 0)

        def padbody(j, carry):
            coef_ref[j, 0] = 0.0
            coef_ref[j, 1] = 0.0
            coef_ref[j, 2] = BIG
            return carry
        lax.fori_loop(NUM_BINS, PAD_BINS, padbody, 0)

    ta = ta_ref[...]
    tb = tb_ref[...]

    def score(j):
        return coef_ref[j, 2] + ta * coef_ref[j, 0] + tb * coef_ref[j, 1]

    def body(g, carry):
        mv, mi = carry
        base = g * GROUP
        vals = [score(base + k) for k in range(GROUP)]
        idxs = list(range(GROUP))
        while len(vals) > 1:
            nv, ni = [], []
            for i in range(0, len(vals), 2):
                take = vals[i + 1] < vals[i]
                nv.append(jnp.where(take, vals[i + 1], vals[i]))
                ni.append(jnp.where(take, idxs[i + 1], idxs[i]))
            vals, idxs = nv, ni
        jwin = idxs[0] + base
        upd = vals[0] < mv
        return jnp.where(upd, vals[0], mv), jnp.where(upd, jwin, mi)

    mv0 = jnp.full((BLOCK_ROWS, 128), BIG, jnp.float32)
    mi0 = jnp.zeros((BLOCK_ROWS, 128), jnp.int32)
    _, mi = lax.fori_loop(0, PAD_BINS // GROUP, body, (mv0, mi0), unroll=1)
    bins_ref[...] = mi
    l2_ref[...] = (pa_ref[...] - ta) ** 2 + (pb_ref[...] - tb) ** 2


def _run_argmin_l2(ta, tb, pa, pb, centers, rows):
    blk = pl.BlockSpec((BLOCK_ROWS, 128), lambda i: (i, 0))
    return pl.pallas_call(
        _argmin_l2_body,
        grid=(rows // BLOCK_ROWS,),
        in_specs=[
            pl.BlockSpec(memory_space=pltpu.SMEM),
            blk, blk, blk, blk,
        ],
        out_specs=[blk, blk],
        out_shape=[
            jax.ShapeDtypeStruct((rows, 128), jnp.int32),
            jax.ShapeDtypeStruct((rows, 128), jnp.float32),
        ],
        scratch_shapes=[pltpu.SMEM((PAD_BINS, 3), jnp.float32)],
        compiler_params=pltpu.CompilerParams(
            dimension_semantics=("arbitrary",),
        ),
    )(centers, ta, tb, pa, pb)


def _make_sc_gather_reduce(n):
    info = plsc.get_sparse_core_info()
    nc, ns, L = info.num_cores, info.num_subcores, info.num_lanes
    nw = nc * ns
    chunk = n // nw

    mesh = plsc.VectorSubcoreMesh(core_axis_name="c", subcore_axis_name="s")

    @functools.partial(
        pl.kernel,
        mesh=mesh,
        out_type=jax.ShapeDtypeStruct((nw * L,), jnp.float32),
        scratch_types=[
            pltpu.VMEM((chunk,), jnp.int32),
            pltpu.VMEM((chunk,), jnp.float32),
            pltpu.VMEM((NUM_BINS,), jnp.float32),
            pltpu.VMEM((L,), jnp.float32),
        ],
        compiler_params=pltpu.CompilerParams(needs_layout_passes=False),
    )
    def sc_gather_reduce(bins_hbm, l2_hbm, cw_hbm, out_hbm, idx_v, l2_v, cw_v, acc_v):
        wid = lax.axis_index("s") * nc + lax.axis_index("c")
        base = wid * chunk
        pltpu.sync_copy(cw_hbm, cw_v)
        pltpu.sync_copy(bins_hbm.at[pl.ds(base, chunk)], idx_v)
        pltpu.sync_copy(l2_hbm.at[pl.ds(base, chunk)], l2_v)

        def body(i, acc):
            iv = idx_v[pl.ds(i * L, L)]
            g = plsc.load_gather(cw_v, [iv])
            lv = l2_v[pl.ds(i * L, L)]
            return acc + g * lv

        acc = lax.fori_loop(0, chunk // L, body, jnp.zeros((L,), jnp.float32))
        acc_v[...] = acc
        pltpu.sync_copy(acc_v, out_hbm.at[pl.ds(wid * L, L)])

    return sc_gather_reduce, nw * L


def _finalize_body(p_ref, o_ref):
    o_ref[0, 0] = jnp.sum(p_ref[...]) * (1.0 / N)


def _run_finalize(partials2d):
    out = pl.pallas_call(
        _finalize_body,
        in_specs=[pl.BlockSpec(memory_space=pltpu.MemorySpace.VMEM)],
        out_specs=pl.BlockSpec(memory_space=pltpu.SMEM),
        out_shape=jax.ShapeDtypeStruct((1, 1), jnp.float32),
    )(partials2d)
    return out.reshape(())


def kernel(pred_ab, target_ab, ab_centers, class_weights):
    pa = pred_ab[:, 0, :, :].reshape(ROWS, 128)
    pb = pred_ab[:, 1, :, :].reshape(ROWS, 128)
    ta = target_ab[:, 0, :, :].reshape(ROWS, 128)
    tb = target_ab[:, 1, :, :].reshape(ROWS, 128)

    half_rows = ROWS // 2
    half_n = half_rows * 128
    sc_fn, npart = _make_sc_gather_reduce(half_n)

    parts = []
    for lo in (0, half_rows):
        sl = slice(lo, lo + half_rows)
        bins, l2 = _run_argmin_l2(ta[sl], tb[sl], pa[sl], pb[sl],
                                  ab_centers, half_rows)
        parts.append(sc_fn(bins.reshape(half_n), l2.reshape(half_n),
                           class_weights))

    partials = jnp.concatenate(parts)
    return _run_finalize(partials.reshape(2 * npart // 128, 128))


# speedup vs baseline: 1.0756x; 1.0756x over previous
import functools

import jax
import jax.numpy as jnp
from jax import lax
from jax.experimental import pallas as pl
from jax.experimental.pallas import tpu as pltpu
from jax.experimental.pallas import tpu_sc as plsc

N = 8 * 224 * 224
ROWS = N // 128
BLOCK_ROWS = 64
GRID = ROWS // BLOCK_ROWS
NUM_BINS = 313
PAD_BINS = 320
GROUP = 16
BIG = 3.0e38


def _argmin_l2_body(centers_ref, ta_ref, tb_ref, pa_ref, pb_ref, bins_ref, l2_ref,
                    coef_ref):
    @pl.when(pl.program_id(0) == 0)
    def _prep():
        def pbody(j, carry):
            c_a = centers_ref[j, 0]
            c_b = centers_ref[j, 1]
            coef_ref[j, 0] = c_a * -256.0
            coef_ref[j, 1] = c_b * -256.0
            coef_ref[j, 2] = c_a * c_a + c_b * c_b
            return carry
        lax.fori_loop(0, NUM_BINS, pbody, 0)

        def padbody(j, carry):
            coef_ref[j, 0] = 0.0
            coef_ref[j, 1] = 0.0
            coef_ref[j, 2] = BIG
            return carry
        lax.fori_loop(NUM_BINS, PAD_BINS, padbody, 0)

    ta = ta_ref[...]
    tb = tb_ref[...]

    def score(j):
        return coef_ref[j, 2] + ta * coef_ref[j, 0] + tb * coef_ref[j, 1]

    def body(g, carry):
        mv, mi = carry
        base = g * GROUP
        vals = [score(base + k) for k in range(GROUP)]
        idxs = list(range(GROUP))
        while len(vals) > 1:
            nv, ni = [], []
            for i in range(0, len(vals), 2):
                take = vals[i + 1] < vals[i]
                nv.append(jnp.where(take, vals[i + 1], vals[i]))
                ni.append(jnp.where(take, idxs[i + 1], idxs[i]))
            vals, idxs = nv, ni
        jwin = idxs[0] + base
        upd = vals[0] < mv
        return jnp.where(upd, vals[0], mv), jnp.where(upd, jwin, mi)

    mv0 = jnp.full((BLOCK_ROWS, 128), BIG, jnp.float32)
    mi0 = jnp.zeros((BLOCK_ROWS, 128), jnp.int32)
    _, mi = lax.fori_loop(0, PAD_BINS // GROUP, body, (mv0, mi0), unroll=1)
    bins_ref[...] = mi
    l2_ref[...] = (pa_ref[...] - ta) ** 2 + (pb_ref[...] - tb) ** 2


def _run_argmin_l2(ta, tb, pa, pb, centers, rows):
    blk = pl.BlockSpec((BLOCK_ROWS, 128), lambda i: (i, 0))
    return pl.pallas_call(
        _argmin_l2_body,
        grid=(rows // BLOCK_ROWS,),
        in_specs=[
            pl.BlockSpec(memory_space=pltpu.SMEM),
            blk, blk, blk, blk,
        ],
        out_specs=[blk, blk],
        out_shape=[
            jax.ShapeDtypeStruct((rows, 128), jnp.int32),
            jax.ShapeDtypeStruct((rows, 128), jnp.float32),
        ],
        scratch_shapes=[pltpu.SMEM((PAD_BINS, 3), jnp.float32)],
        compiler_params=pltpu.CompilerParams(
            dimension_semantics=("arbitrary",),
        ),
    )(centers, ta, tb, pa, pb)


def _make_sc_gather_reduce(n):
    info = plsc.get_sparse_core_info()
    nc, ns, L = info.num_cores, info.num_subcores, info.num_lanes
    nw = nc * ns
    chunk = n // nw

    mesh = plsc.VectorSubcoreMesh(core_axis_name="c", subcore_axis_name="s")

    @functools.partial(
        pl.kernel,
        mesh=mesh,
        out_type=jax.ShapeDtypeStruct((nw * L,), jnp.float32),
        scratch_types=[
            pltpu.VMEM((chunk,), jnp.int32),
            pltpu.VMEM((chunk,), jnp.float32),
            pltpu.VMEM((NUM_BINS,), jnp.float32),
            pltpu.VMEM((L,), jnp.float32),
        ],
        compiler_params=pltpu.CompilerParams(needs_layout_passes=False),
    )
    def sc_gather_reduce(bins_hbm, l2_hbm, cw_hbm, out_hbm, idx_v, l2_v, cw_v, acc_v):
        wid = lax.axis_index("s") * nc + lax.axis_index("c")
        base = wid * chunk
        pltpu.sync_copy(cw_hbm, cw_v)
        pltpu.sync_copy(bins_hbm.at[pl.ds(base, chunk)], idx_v)
        pltpu.sync_copy(l2_hbm.at[pl.ds(base, chunk)], l2_v)

        def body(i, acc):
            iv = idx_v[pl.ds(i * L, L)]
            g = plsc.load_gather(cw_v, [iv])
            lv = l2_v[pl.ds(i * L, L)]
            return acc + g * lv

        acc = lax.fori_loop(0, chunk // L, body, jnp.zeros((L,), jnp.float32))
        acc_v[...] = acc
        pltpu.sync_copy(acc_v, out_hbm.at[pl.ds(wid * L, L)])

    return sc_gather_reduce, nw * L


def _finalize_body(p_ref, o_ref):
    o_ref[0, 0] = jnp.sum(p_ref[...]) * (1.0 / N)


def _run_finalize(partials2d):
    out = pl.pallas_call(
        _finalize_body,
        in_specs=[pl.BlockSpec(memory_space=pltpu.MemorySpace.VMEM)],
        out_specs=pl.BlockSpec(memory_space=pltpu.SMEM),
        out_shape=jax.ShapeDtypeStruct((1, 1), jnp.float32),
    )(partials2d)
    return out.reshape(())


def kernel(pred_ab, target_ab, ab_centers, class_weights):
    pa = pred_ab[:, 0, :, :].reshape(ROWS, 128)
    pb = pred_ab[:, 1, :, :].reshape(ROWS, 128)
    ta = target_ab[:, 0, :, :].reshape(ROWS, 128)
    tb = target_ab[:, 1, :, :].reshape(ROWS, 128)

    bins, l2 = _run_argmin_l2(ta, tb, pa, pb, ab_centers, ROWS)

    sc_fn, npart = _make_sc_gather_reduce(N)
    partials = sc_fn(bins.reshape(N), l2.reshape(N), class_weights)

    return _run_finalize(partials.reshape(npart // 128, 128))
